# Initial kernel scaffold; baseline (speedup 1.0000x reference)
import jax
import jax.numpy as jnp
from jax.experimental import pallas as pl


def kernel(x, edge_index, weight, bias):
    raise NotImplementedError("write your pallas kernel here")


# speedup vs baseline: 7.6104x; 7.6104x over previous
import functools

import jax
import jax.numpy as jnp
from jax import lax
from jax.experimental import pallas as pl
from jax.experimental.pallas import tpu as pltpu
from jax.experimental.pallas import tpu_sc as plsc

N = 10000
NP = 10240
C = 128
B = 4
E = 320000
PADE = 327680
ROWS = PADE // 128
NDUMMY = NP - N
NB = 1280
STRIPE = NP // 16


def _sc_mesh():
    return plsc.VectorSubcoreMesh(core_axis_name="c", subcore_axis_name="s")


def _deg_body(row2d, zeros1d, ones128, deg_out, idx_buf, ones_buf, acc):
    c = lax.axis_index("c")
    s = lax.axis_index("s")
    wid = c * 16 + s
    pltpu.sync_copy(zeros1d, acc.at[pl.ds(s * STRIPE, STRIPE)])
    pltpu.sync_copy(ones128, ones_buf)
    pltpu.sync_copy(row2d.at[pl.ds(wid * (ROWS // 32), ROWS // 32)], idx_buf)
    plsc.subcore_barrier()

    def win(j, carry):
        pltpu.sync_copy(ones_buf, acc.at[idx_buf.at[j]], add=True)
        return carry

    lax.fori_loop(0, ROWS // 32, win, 0)
    plsc.subcore_barrier()
    pltpu.sync_copy(acc.at[pl.ds(s * STRIPE, STRIPE)],
                    deg_out.at[c].at[pl.ds(s * STRIPE, STRIPE)])


def _deg_kernel(row2d, zeros1d, ones128):
    f = pl.kernel(
        _deg_body,
        out_type=jax.ShapeDtypeStruct((2, NP), jnp.float32),
        mesh=_sc_mesh(),
        scratch_types=[
            pltpu.VMEM((ROWS // 32, 128), jnp.int32),
            pltpu.VMEM((128,), jnp.float32),
            pltpu.VMEM_SHARED((NP,), jnp.float32),
        ],
    )
    return f(row2d, zeros1d, ones128)


IDXCH = 32


def _spmm_body(g, col2d, row2d, zeros2d, s_out, colb, rowb, gbuf, acc, sem):
    c = lax.axis_index("c")
    s = lax.axis_index("s")
    nwin = ROWS // 16
    for r in range(2):
        b = 2 * r + c
        pltpu.sync_copy(zeros2d, acc.at[pl.ds(s * STRIPE, STRIPE)])
        plsc.subcore_barrier()

        for ch in range(nwin // IDXCH):
            base = s * nwin + ch * IDXCH
            pltpu.sync_copy(col2d.at[pl.ds(base, IDXCH)], colb)
            pltpu.sync_copy(row2d.at[pl.ds(base, IDXCH)], rowb)

            def win(j, carry):
                pltpu.async_copy(g.at[b].at[colb.at[j]], gbuf, sem).wait()
                pltpu.sync_copy(gbuf, acc.at[rowb.at[j]], add=True)
                return carry

            lax.fori_loop(0, IDXCH, win, 0)
        plsc.subcore_barrier()
        for k in range(STRIPE // 128):
            pltpu.sync_copy(
                acc.at[pl.ds(s * STRIPE + k * 128, 128)],
                s_out.at[b].at[pl.ds(s * STRIPE + k * 128, 128)])
        plsc.subcore_barrier()


def _spmm(g, col2d, row2d, zeros2d):
    f = pl.kernel(
        _spmm_body,
        out_type=jax.ShapeDtypeStruct((B, NP, C), jnp.float32),
        mesh=_sc_mesh(),
        scratch_types=[
            pltpu.VMEM((IDXCH, 128), jnp.int32),
            pltpu.VMEM((IDXCH, 128), jnp.int32),
            pltpu.VMEM((128, C), jnp.float32),
            pltpu.VMEM_SHARED((NP, C), jnp.float32),
            pltpu.SemaphoreType.DMA,
        ],
    )
    return f(g, col2d, row2d, zeros2d)


def _scale_x_body(deg_ref, x_ref, o_ref):
    d = deg_ref[0, :] + deg_ref[1, :]
    dis = jnp.where(d > 0.0, lax.rsqrt(jnp.where(d > 0.0, d, 1.0)), 0.0)
    o_ref[...] = x_ref[...] * dis[None, :, None]


def _scale_s_body(deg_ref, s_ref, o_ref):
    d = deg_ref[0, :] + deg_ref[1, :]
    scale = jnp.where(d > 0.0, -1.0 / jnp.where(d > 0.0, d, 1.0), 0.0)
    o_ref[...] = s_ref[...] * scale[None, :, None]


def _scale(body, deg2, arr):
    return pl.pallas_call(
        body,
        grid=(B, NP // NB),
        in_specs=[
            pl.BlockSpec((2, NB), lambda b, n: (0, n)),
            pl.BlockSpec((1, NB, C), lambda b, n: (b, n, 0)),
        ],
        out_specs=pl.BlockSpec((1, NB, C), lambda b, n: (b, n, 0)),
        out_shape=jax.ShapeDtypeStruct((B, NP, C), jnp.float32),
    )(deg2, arr)


def _final_body(deg_ref, x_ref, s1_ref, s2_ref, w_ref, b_ref, o_ref):
    d = deg_ref[0, :] + deg_ref[1, :]
    dis = jnp.where(d > 0.0, lax.rsqrt(jnp.where(d > 0.0, d, 1.0)), 0.0)
    xb = x_ref[0]
    t1 = (-dis)[:, None] * s1_ref[0]
    t2 = (-2.0 * dis)[:, None] * s2_ref[0]
    o_ref[0] = (jnp.dot(xb, w_ref[0] - w_ref[2], preferred_element_type=jnp.float32)
                + jnp.dot(t1, w_ref[1], preferred_element_type=jnp.float32)
                + jnp.dot(t2, w_ref[2], preferred_element_type=jnp.float32)
                + b_ref[...])


def _final(deg2, xp, s1, s2, weight, bias2d):
    return pl.pallas_call(
        _final_body,
        grid=(B, NP // NB),
        in_specs=[
            pl.BlockSpec((2, NB), lambda b, n: (0, n)),
            pl.BlockSpec((1, NB, C), lambda b, n: (b, n, 0)),
            pl.BlockSpec((1, NB, C), lambda b, n: (b, n, 0)),
            pl.BlockSpec((1, NB, C), lambda b, n: (b, n, 0)),
            pl.BlockSpec((3, C, C), lambda b, n: (0, 0, 0)),
            pl.BlockSpec((1, C), lambda b, n: (0, 0)),
        ],
        out_specs=pl.BlockSpec((1, NB, C), lambda b, n: (b, n, 0)),
        out_shape=jax.ShapeDtypeStruct((B, NP, C), jnp.float32),
    )(deg2, xp, s1, s2, weight, bias2d)


def kernel(x, edge_index, weight, bias):
    row = edge_index[0].astype(jnp.int32)
    col = edge_index[1].astype(jnp.int32)
    fixed = jnp.where(row == col, N + (row % NDUMMY), row)
    padn = PADE - E
    spread = N + (jnp.arange(padn, dtype=jnp.int32) % NDUMMY)
    row2d = jnp.concatenate([fixed, spread]).reshape(ROWS, 128)
    col2d = jnp.concatenate([col, spread]).reshape(ROWS, 128)
    xp = jnp.pad(x, ((0, 0), (0, NP - N), (0, 0)))
    zeros1d = jnp.zeros((STRIPE,), jnp.float32)
    zeros2d = jnp.zeros((STRIPE, C), jnp.float32)
    ones128 = jnp.ones((128,), jnp.float32)

    deg2 = _deg_kernel(row2d, zeros1d, ones128)
    g1 = _scale(_scale_x_body, deg2, xp)
    s1 = _spmm(g1, col2d, row2d, zeros2d)
    g2 = _scale(_scale_s_body, deg2, s1)
    s2 = _spmm(g2, col2d, row2d, zeros2d)
    outp = _final(deg2, xp, s1, s2, weight, jnp.reshape(bias, (1, C)))
    return outp[:, :N, :]


# speedup vs baseline: 11.6553x; 1.5315x over previous
import functools

import jax
import jax.numpy as jnp
from jax import lax
from jax.experimental import pallas as pl
from jax.experimental.pallas import tpu as pltpu
from jax.experimental.pallas import tpu_sc as plsc

N = 10000
NP = 10240
C = 128
B = 4
E = 320000
PADE = 327680
ROWS = PADE // 128
NDUMMY = NP - N
NB = 1280
STRIPE = NP // 16


def _sc_mesh():
    return plsc.VectorSubcoreMesh(core_axis_name="c", subcore_axis_name="s")


def _deg_body(row2d, zeros1d, ones128, deg_out, idx_buf, ones_buf, acc):
    c = lax.axis_index("c")
    s = lax.axis_index("s")
    wid = c * 16 + s
    pltpu.sync_copy(zeros1d, acc.at[pl.ds(s * STRIPE, STRIPE)])
    pltpu.sync_copy(ones128, ones_buf)
    pltpu.sync_copy(row2d.at[pl.ds(wid * (ROWS // 32), ROWS // 32)], idx_buf)
    plsc.subcore_barrier()

    def win(j, carry):
        pltpu.sync_copy(ones_buf, acc.at[idx_buf.at[j]], add=True)
        return carry

    lax.fori_loop(0, ROWS // 32, win, 0)
    plsc.subcore_barrier()
    pltpu.sync_copy(acc.at[pl.ds(s * STRIPE, STRIPE)],
                    deg_out.at[c].at[pl.ds(s * STRIPE, STRIPE)])


def _deg_kernel(row2d, zeros1d, ones128):
    f = pl.kernel(
        _deg_body,
        out_type=jax.ShapeDtypeStruct((2, NP), jnp.float32),
        mesh=_sc_mesh(),
        scratch_types=[
            pltpu.VMEM((ROWS // 32, 128), jnp.int32),
            pltpu.VMEM((128,), jnp.float32),
            pltpu.VMEM_SHARED((NP,), jnp.float32),
        ],
    )
    return f(row2d, zeros1d, ones128)


IDXCH = 32


def _spmm_body(g, col2d, row2d, zeros2d, s_out,
               colb, rowb, gb0, gb1, acc, sg0, sg1, ss0, ss1):
    c = lax.axis_index("c")
    s = lax.axis_index("s")
    nwin = ROWS // 16
    for r in range(2):
        b = 2 * r + c
        gsrc = g.at[b]
        pltpu.sync_copy(zeros2d, acc.at[pl.ds(s * STRIPE, STRIPE)])
        plsc.subcore_barrier()

        def chunk(ch, carry):
            base = s * nwin + ch * IDXCH
            pltpu.sync_copy(col2d.at[pl.ds(base, IDXCH)], colb)
            pltpu.sync_copy(row2d.at[pl.ds(base, IDXCH)], rowb)
            gds = [pltpu.async_copy(gsrc.at[colb.at[0]], gb0, sg0),
                   pltpu.async_copy(gsrc.at[colb.at[1]], gb1, sg1)]
            sds = [None, None]
            for j in range(IDXCH):
                p = j & 1
                gb, sg, ss = (gb0, sg0, ss0) if p == 0 else (gb1, sg1, ss1)
                gds[p].wait()
                sds[p] = pltpu.async_copy(gb, acc.at[rowb.at[j]], ss, add=True)
                if j + 2 < IDXCH:
                    sds[p].wait()
                    gds[p] = pltpu.async_copy(gsrc.at[colb.at[j + 2]], gb, sg)
            sds[0].wait()
            sds[1].wait()
            return carry

        lax.fori_loop(0, nwin // IDXCH, chunk, 0)
        plsc.subcore_barrier()
        for k in range(STRIPE // 128):
            pltpu.sync_copy(
                acc.at[pl.ds(s * STRIPE + k * 128, 128)],
                s_out.at[b].at[pl.ds(s * STRIPE + k * 128, 128)])
        plsc.subcore_barrier()


def _spmm(g, col2d, row2d, zeros2d):
    f = pl.kernel(
        _spmm_body,
        out_type=jax.ShapeDtypeStruct((B, NP, C), jnp.float32),
        mesh=_sc_mesh(),
        scratch_types=[
            pltpu.VMEM((IDXCH, 128), jnp.int32),
            pltpu.VMEM((IDXCH, 128), jnp.int32),
            pltpu.VMEM((128, C), jnp.float32),
            pltpu.VMEM((128, C), jnp.float32),
            pltpu.VMEM_SHARED((NP, C), jnp.float32),
            pltpu.SemaphoreType.DMA,
            pltpu.SemaphoreType.DMA,
            pltpu.SemaphoreType.DMA,
            pltpu.SemaphoreType.DMA,
        ],
    )
    return f(g, col2d, row2d, zeros2d)


def _scale_x_body(deg_ref, x_ref, o_ref):
    d = deg_ref[0, :] + deg_ref[1, :]
    dis = jnp.where(d > 0.0, lax.rsqrt(jnp.where(d > 0.0, d, 1.0)), 0.0)
    o_ref[...] = x_ref[...] * dis[None, :, None]


def _scale_s_body(deg_ref, s_ref, o_ref):
    d = deg_ref[0, :] + deg_ref[1, :]
    scale = jnp.where(d > 0.0, -1.0 / jnp.where(d > 0.0, d, 1.0), 0.0)
    o_ref[...] = s_ref[...] * scale[None, :, None]


def _scale(body, deg2, arr):
    return pl.pallas_call(
        body,
        grid=(B, NP // NB),
        in_specs=[
            pl.BlockSpec((2, NB), lambda b, n: (0, n)),
            pl.BlockSpec((1, NB, C), lambda b, n: (b, n, 0)),
        ],
        out_specs=pl.BlockSpec((1, NB, C), lambda b, n: (b, n, 0)),
        out_shape=jax.ShapeDtypeStruct((B, NP, C), jnp.float32),
    )(deg2, arr)


def _final_body(deg_ref, x_ref, s1_ref, s2_ref, w_ref, b_ref, o_ref):
    d = deg_ref[0, :] + deg_ref[1, :]
    dis = jnp.where(d > 0.0, lax.rsqrt(jnp.where(d > 0.0, d, 1.0)), 0.0)
    xb = x_ref[0]
    t1 = (-dis)[:, None] * s1_ref[0]
    t2 = (-2.0 * dis)[:, None] * s2_ref[0]
    o_ref[0] = (jnp.dot(xb, w_ref[0] - w_ref[2], preferred_element_type=jnp.float32)
                + jnp.dot(t1, w_ref[1], preferred_element_type=jnp.float32)
                + jnp.dot(t2, w_ref[2], preferred_element_type=jnp.float32)
                + b_ref[...])


def _final(deg2, xp, s1, s2, weight, bias2d):
    return pl.pallas_call(
        _final_body,
        grid=(B, NP // NB),
        in_specs=[
            pl.BlockSpec((2, NB), lambda b, n: (0, n)),
            pl.BlockSpec((1, NB, C), lambda b, n: (b, n, 0)),
            pl.BlockSpec((1, NB, C), lambda b, n: (b, n, 0)),
            pl.BlockSpec((1, NB, C), lambda b, n: (b, n, 0)),
            pl.BlockSpec((3, C, C), lambda b, n: (0, 0, 0)),
            pl.BlockSpec((1, C), lambda b, n: (0, 0)),
        ],
        out_specs=pl.BlockSpec((1, NB, C), lambda b, n: (b, n, 0)),
        out_shape=jax.ShapeDtypeStruct((B, NP, C), jnp.float32),
    )(deg2, xp, s1, s2, weight, bias2d)


def kernel(x, edge_index, weight, bias):
    row = edge_index[0].astype(jnp.int32)
    col = edge_index[1].astype(jnp.int32)
    fixed = jnp.where(row == col, N + (row % NDUMMY), row)
    padn = PADE - E
    spread = N + (jnp.arange(padn, dtype=jnp.int32) % NDUMMY)
    row2d = jnp.concatenate([fixed, spread]).reshape(ROWS, 128)
    col2d = jnp.concatenate([col, spread]).reshape(ROWS, 128)
    xp = jnp.pad(x, ((0, 0), (0, NP - N), (0, 0)))
    zeros1d = jnp.zeros((STRIPE,), jnp.float32)
    zeros2d = jnp.zeros((STRIPE, C), jnp.float32)
    ones128 = jnp.ones((128,), jnp.float32)

    deg2 = _deg_kernel(row2d, zeros1d, ones128)
    g1 = _scale(_scale_x_body, deg2, xp)
    s1 = _spmm(g1, col2d, row2d, zeros2d)
    g2 = _scale(_scale_s_body, deg2, s1)
    s2 = _spmm(g2, col2d, row2d, zeros2d)
    outp = _final(deg2, xp, s1, s2, weight, jnp.reshape(bias, (1, C)))
    return outp[:, :N, :]
